# Initial kernel scaffold; baseline (speedup 1.0000x reference)
import jax
import jax.numpy as jnp
from jax.experimental import pallas as pl


def kernel(x, edge_index, batch, stats, params):
    raise NotImplementedError("write your pallas kernel here")


# speedup vs baseline: 1.3591x; 1.3591x over previous
import functools

import jax
import jax.numpy as jnp
import numpy as np
from jax import lax
from jax.experimental import pallas as pl
from jax.experimental.pallas import tpu as pltpu
from jax.experimental.pallas import tpu_sc as plsc

N = 10000
E = 320000
B = 200
DIN = 128
HID = 64
LAT = 32
HDEC = 256
NMAX = 50
NPAIR = NMAX * (NMAX - 1) // 2

NPAD = 10240
NBLK = 8
BLKR = NPAD // NBLK
BPAD = 256
NPAIR_PAD = 1280
ADJ = NMAX * NMAX
ADJ_PAD = 2560

SC_CORES = 2
SC_TILES = 16
WORKERS = SC_CORES * SC_TILES
EK = 128

BN_S = float(np.float32(1.0) / np.sqrt(np.float32(1.0) + np.float32(1e-5)))

_iu, _ju = np.triu_indices(NMAX, k=1)
_P = np.zeros((NPAIR_PAD, ADJ_PAD), np.float32)
_P[np.arange(NPAIR), _iu * NMAX + _ju] = 1.0
_P[np.arange(NPAIR), _ju * NMAX + _iu] = 1.0

_SC_MESH = dict(core_axis_name="c", subcore_axis_name="s")
_SC_PARAMS = pltpu.CompilerParams(use_tc_tiling_on_sc=False,
                                 needs_layout_passes=False)


def _wid():
    return lax.axis_index("c") * SC_TILES + lax.axis_index("s")


def _sc_partition(src, dst, nelem, chunk, rsize, maxe, pad_src=N):
    nch = nelem // chunk
    assert nch * chunk == nelem and chunk % 16 == 0

    @functools.partial(
        pl.kernel,
        mesh=plsc.VectorSubcoreMesh(**_SC_MESH),
        out_type=(jax.ShapeDtypeStruct((WORKERS, maxe), jnp.int32),
                  jax.ShapeDtypeStruct((WORKERS, maxe), jnp.int32),
                  jax.ShapeDtypeStruct((WORKERS, 16), jnp.int32)),
        scratch_types=[
            pltpu.VMEM((chunk,), jnp.int32),
            pltpu.VMEM((chunk,), jnp.int32),
            pltpu.VMEM((maxe + 16,), jnp.int32),
            pltpu.VMEM((maxe + 16,), jnp.int32),
            pltpu.VMEM((16,), jnp.int32),
            pltpu.VMEM((128,), jnp.int32),
        ],
        compiler_params=_SC_PARAMS,
    )
    def part(src_h, dst_h, psrc_h, pdst_h, pcnt_h, sstage, dstage, sbuf,
             dbuf, cbuf, shbuf):
        w = _wid()
        lo = w * rsize
        hi = lo + rsize

        def zsh(i, carry):
            shbuf[pl.ds(i * 16, 16)] = jnp.zeros((16,), jnp.int32)
            return carry

        lax.fori_loop(0, 8, zsh, 0)

        def fill(i, carry):
            sbuf[pl.ds(i * 16, 16)] = jnp.full((16,), pad_src, jnp.int32)
            dbuf[pl.ds(i * 16, 16)] = jnp.full((16,), lo, jnp.int32)
            return carry

        lax.fori_loop(0, (maxe + 16) // 16, fill, 0)

        def scan_chunk(c, cnt):
            pltpu.sync_copy(src_h.at[pl.ds(c * chunk, chunk)], sstage)
            pltpu.sync_copy(dst_h.at[pl.ds(c * chunk, chunk)], dstage)

            def inner(i, cntv):
                sl = pl.ds(i * 16, 16)
                d = dstage[sl]
                s = sstage[sl]
                lane = lax.iota(jnp.int32, 16)
                m = (d >= lo) & (d < hi) & (cntv <= maxe - 16)
                pos = jnp.zeros((16,), jnp.int32)
                for j in range(16):
                    pc = plsc.all_reduce_population_count(m & (lane <= j))
                    pos = pos + jnp.where(lane == j, pc, 0)
                tgt = jnp.where(m, cntv + pos - 1, maxe)
                plsc.store_scatter(dbuf, [tgt], d)
                plsc.store_scatter(sbuf, [tgt], s)
                return cntv + plsc.all_reduce_population_count(m)

            return lax.fori_loop(0, chunk // 16, inner, cnt)

        cnt = lax.fori_loop(0, nch, scan_chunk, jnp.zeros((16,), jnp.int32))
        cbuf[pl.ds(0, 16)] = cnt
        pltpu.sync_copy(sbuf.at[pl.ds(0, maxe)], psrc_h.at[w])
        pltpu.sync_copy(dbuf.at[pl.ds(0, maxe)], pdst_h.at[w])
        pltpu.sync_copy(cbuf, pcnt_h.at[w])

    return part(src, dst)


def _sc_ordered_segsum(table, psrc, pdst, pcnt, rsize, wine=1 << 29):
    width = table.shape[1]
    maxe = psrc.shape[1]

    @functools.partial(
        pl.kernel,
        mesh=plsc.VectorSubcoreMesh(**_SC_MESH),
        out_type=jax.ShapeDtypeStruct((WORKERS * rsize, width), jnp.float32),
        scratch_types=[
            pltpu.VMEM((maxe,), jnp.int32),
            pltpu.VMEM((maxe,), jnp.int32),
            pltpu.VMEM((WORKERS, 16), jnp.int32),
            pltpu.VMEM((EK, width), jnp.float32),
            pltpu.VMEM((rsize + 2, width), jnp.float32),
            pltpu.SemaphoreType.DMA,
        ],
        compiler_params=_SC_PARAMS,
    )
    def seg(table_h, psrc_h, pdst_h, pcnt_h, out_h, sbuf, dbuf, cbuf, rows,
            acc, sem):
        w = _wid()
        lo = w * rsize
        pltpu.sync_copy(psrc_h.at[w], sbuf)
        pltpu.sync_copy(pdst_h.at[w], dbuf)
        pltpu.sync_copy(pcnt_h, cbuf)

        def zrow(r, carry):
            for c in range(width // 16):
                acc[r, pl.ds(c * 16, 16)] = jnp.zeros((16,), jnp.float32)
            return carry

        lax.fori_loop(0, rsize + 2, zrow, 0)

        def pre(i, st):
            sw, cw = st
            ci = cbuf[i, pl.ds(0, 16)][0]
            return (sw + jnp.where(i < w, ci, 0),
                    cw + jnp.where(i == w, ci, 0))

        s_w, cnt = lax.fori_loop(0, WORKERS, pre, (jnp.int32(0), jnp.int32(0)))
        nch = (cnt + EK - 1) >> 7

        def count_lt(mid):
            def cb(j, accv):
                base2 = j * EK
                for grp in range(EK // 16):
                    posv = base2 + grp * 16 + lax.iota(jnp.int32, 16)
                    dvv = dbuf[pl.ds(base2 + grp * 16, 16)]
                    ok = (dvv < mid) & (posv < cnt)
                    accv = accv + plsc.all_reduce_population_count(ok)
                return accv
            return lax.fori_loop(0, nch, cb, jnp.zeros((16,), jnp.int32))

        def bound(k, slot):
            jb = (s_w // wine + 1 + k) * wine - s_w
            lov = jnp.full((16,), lo, jnp.int32)
            hiv = jnp.full((16,), lo + rsize - 1, jnp.int32)
            for _ in range(9):
                midv = (lov + hiv) >> 1
                cm = count_lt(midv + 1)
                gt = cm > jb
                hiv = jnp.where(gt, midv, hiv)
                lov = jnp.where(gt, lov, midv + 1)
            rho = jb - count_lt(lov)
            cbuf[slot, pl.ds(0, 16)] = jnp.where(jnp.full((16,), jb < cnt,
                                                          jnp.bool_),
                                                 lov - lo, -1)
            cbuf[slot + 2, pl.ds(0, 16)] = rho

        bound(0, 0)
        bound(1, 1)
        r1 = cbuf[0, pl.ds(0, 16)][0]
        r2 = cbuf[1, pl.ds(0, 16)][0]
        rho1 = cbuf[2, pl.ds(0, 16)][0]
        rho2 = cbuf[3, pl.ds(0, 16)][0]

        def chunk_body(j, carry):
            base = j * EK
            pltpu.async_copy(table_h.at[sbuf.at[pl.ds(base, EK)]], rows,
                             sem).wait()
            c1, c2 = carry
            for grp in range(EK // 16):
                dv = dbuf[pl.ds(base + grp * 16, 16)]
                for k in range(16):
                    r = dv[k] - lo
                    is1 = r == r1
                    is2 = r == r2
                    rr = jnp.where(is1 & (c1 >= rho1), rsize, r)
                    rr = jnp.where(is2 & (c2 >= rho2), rsize + 1, rr)
                    c1 = c1 + is1.astype(jnp.int32)
                    c2 = c2 + is2.astype(jnp.int32)
                    for c in range(width // 16):
                        sl = pl.ds(c * 16, 16)
                        acc[rr, sl] = acc[rr, sl] + rows[grp * 16 + k, sl]
            return (c1, c2)

        lax.fori_loop(0, nch, chunk_body, (jnp.int32(0), jnp.int32(0)))
        r1f = jnp.maximum(r1, 0)
        r2f = jnp.maximum(r2, 0)
        for c in range(width // 16):
            sl = pl.ds(c * 16, 16)
            acc[r1f, sl] = acc[r1f, sl] + acc[rsize, sl]
        for c in range(width // 16):
            sl = pl.ds(c * 16, 16)
            acc[r2f, sl] = acc[r2f, sl] + acc[rsize + 1, sl]
        pltpu.sync_copy(acc.at[pl.ds(0, rsize)], out_h.at[pl.ds(lo, rsize)])

    return seg(table, psrc, pdst, pcnt)


def _lrelu(v):
    return jnp.where(v >= 0, v, 0.2 * v)


def _combine_body(h_ref, agg_ref, w1_ref, b1_ref, g_ref, bb_ref, w2_ref,
                  b2_ref, o_ref):
    z = h_ref[...] + agg_ref[...]
    m = _lrelu(jnp.dot(z, w1_ref[...], preferred_element_type=jnp.float32) + b1_ref[...])
    m = (m * BN_S) * g_ref[...] + bb_ref[...]
    o = _lrelu(jnp.dot(m, w2_ref[...], preferred_element_type=jnp.float32) + b2_ref[...])
    row = pl.program_id(0) * BLKR + lax.broadcasted_iota(jnp.int32, (BLKR, 1), 0)
    o_ref[...] = jnp.where(row < N, o, 0.0)


def _vec_specs(n):
    return [pl.BlockSpec((1, HID), lambda i: (0, 0)) for _ in range(n)]


def _combine(h, agg, w1, b1, g, bb, w2, b2):
    din = h.shape[1]
    return pl.pallas_call(
        _combine_body,
        grid=(NBLK,),
        in_specs=[pl.BlockSpec((BLKR, din), lambda i: (i, 0)),
                  pl.BlockSpec((BLKR, din), lambda i: (i, 0)),
                  pl.BlockSpec((din, HID), lambda i: (0, 0))]
                 + _vec_specs(3)
                 + [pl.BlockSpec((HID, HID), lambda i: (0, 0))]
                 + _vec_specs(1),
        out_specs=pl.BlockSpec((BLKR, HID), lambda i: (i, 0)),
        out_shape=jax.ShapeDtypeStruct((NPAD, HID), jnp.float32),
    )(h, agg, w1, b1, g, bb, w2, b2)


def _head_body(pooled, statsr, gA, bA, gB, bB, wp, ws, fcb, muw, mub,
               d0a, d0s, d0bias, d1w, d1b, d2e, d2o, b2e, b2o, g0, g1, o_ref):
    dot = functools.partial(jnp.dot, preferred_element_type=jnp.float32)
    a = (pooled[...] * BN_S) * gA[...] + bA[...]
    bst = (statsr[...] * BN_S) * gB[...] + bB[...]
    enc = dot(a, wp[...]) + dot(bst, ws[...]) + fcb[...]
    mu = dot(enc, muw[...]) + mub[...]
    hd = jnp.maximum(dot(mu, d0a[...]) + dot(statsr[...], d0s[...]) + d0bias[...], 0.0)
    h1 = jnp.maximum(dot(hd, d1w[...]) + d1b[...], 0.0)
    l0 = dot(h1, d2e[...]) + b2e[...] + g0[...]
    l1 = dot(h1, d2o[...]) + b2o[...] + g1[...]
    o_ref[...] = (l0 >= l1).astype(jnp.float32)


def kernel(x, edge_index, batch, stats, params):
    f32 = jnp.float32
    convs = params["convs"]

    h = jnp.pad(x, ((0, NPAD - N), (0, 0)))
    statsp = jnp.pad(stats, ((0, BPAD - B), (0, 0)))
    node_iota = jnp.arange(N, dtype=jnp.int32)

    def vrow(v):
        return v.reshape(1, HID)

    esrc, edst, ecnt = _sc_partition(edge_index[0], edge_index[1], E,
                                     chunk=4000, rsize=NPAD // WORKERS,
                                     maxe=12288)
    POOL_WIN = 640
    NW = -(-N // POOL_WIN)
    NKEY = B * NW
    KS1 = 104
    wkey = batch * NW + node_iota // POOL_WIN
    k1 = _sc_partition(node_iota, wkey, N, chunk=2000, rsize=KS1, maxe=1024)
    iota2 = jnp.arange(NKEY, dtype=jnp.int32)
    k2 = _sc_partition(iota2, iota2 // NW, NKEY, chunk=1600,
                       rsize=BPAD // WORKERS, maxe=256, pad_src=NKEY)

    for li in range(3):
        cp = convs[li]
        agg = _sc_ordered_segsum(h, esrc, edst, ecnt, NPAD // WORKERS,
                                 wine=10080)
        h = _combine(h, agg, cp["W1"], vrow(cp["b1"]), vrow(cp["bn_g"]),
                     vrow(cp["bn_b"]), cp["W2"], vrow(cp["b2"]))

    part1 = _sc_ordered_segsum(h, *k1, KS1)
    pooled = _sc_ordered_segsum(part1, *k2, BPAD // WORKERS)

    ur = jax.random.uniform(jax.random.key(42), (B, NPAIR, 2),
                            minval=1e-9, maxval=1.0)
    g = -jnp.log(-jnp.log(ur))
    g0 = jnp.pad(g[:, :, 0], ((0, BPAD - B), (0, NPAIR_PAD - NPAIR)))
    g1 = jnp.pad(g[:, :, 1], ((0, BPAD - B), (0, NPAIR_PAD - NPAIR)))
    d2e = jnp.pad(params["d2W"][:, 0::2], ((0, 0), (0, NPAIR_PAD - NPAIR)))
    d2o = jnp.pad(params["d2W"][:, 1::2], ((0, 0), (0, NPAIR_PAD - NPAIR)))
    b2e = jnp.pad(params["d2b"][0::2], (0, NPAIR_PAD - NPAIR)).reshape(1, NPAIR_PAD)
    b2o = jnp.pad(params["d2b"][1::2], (0, NPAIR_PAD - NPAIR)).reshape(1, NPAIR_PAD)

    xv = pl.pallas_call(
        _head_body,
        out_shape=jax.ShapeDtypeStruct((BPAD, NPAIR_PAD), f32),
    )(pooled, statsp,
      params["enc_bn_g"][:HID].reshape(1, HID), params["enc_bn_b"][:HID].reshape(1, HID),
      params["enc_bn_g"][HID:].reshape(1, 7), params["enc_bn_b"][HID:].reshape(1, 7),
      params["enc_fc_W"][:HID], params["enc_fc_W"][HID:],
      params["enc_fc_b"].reshape(1, HID),
      params["mu_W"], params["mu_b"].reshape(1, LAT),
      params["d0W"][:LAT], params["d0W"][LAT:], params["d0b"].reshape(1, HDEC),
      params["d1W"], params["d1b"].reshape(1, HDEC),
      d2e, d2o, b2e, b2o, g0, g1)

    def _mm_body(x_ref, w_ref, o_ref):
        o_ref[...] = jnp.dot(x_ref[...], w_ref[...], preferred_element_type=jnp.float32)

    adjp = pl.pallas_call(
        _mm_body,
        grid=(4,),
        in_specs=[pl.BlockSpec((BPAD, NPAIR_PAD), lambda j: (0, 0)),
                  pl.BlockSpec((NPAIR_PAD, ADJ_PAD // 4), lambda j: (0, j))],
        out_specs=pl.BlockSpec((BPAD, ADJ_PAD // 4), lambda j: (0, j)),
        out_shape=jax.ShapeDtypeStruct((BPAD, ADJ_PAD), f32),
    )(xv, jnp.asarray(_P))

    return adjp[:B, :ADJ].reshape(B, NMAX, NMAX)
